# baseline (speedup 1.0000x reference)
import functools

import jax
import jax.numpy as jnp
from jax import lax
from jax.experimental import pallas as pl
from jax.experimental.pallas import tpu as pltpu
from jax.experimental.pallas import tpu_sc as plsc

_L = 16


@functools.partial(jax.jit, static_argnums=(2, 3, 4, 5))
def _edge_sc(x2, nbr1, BN, D, K, G):
    NC, NS = 2, 16
    NW = NC * NS
    NPW = BN // NW
    GK = G * K
    n_groups = NPW // G
    NI, NG, NO = 4, 2, 4
    NJ = D // _L

    mesh = plsc.VectorSubcoreMesh(core_axis_name="c", subcore_axis_name="s")

    @functools.partial(
        pl.kernel,
        mesh=mesh,
        out_type=jax.ShapeDtypeStruct((BN * K, 2 * D), jnp.float32),
        scratch_types=[
            pltpu.VMEM((NI, GK), jnp.int32),
            pltpu.VMEM((NG, GK, D), jnp.float32),
            pltpu.VMEM((NG, G, D), jnp.float32),
            pltpu.VMEM((NO, GK, 2 * D), jnp.float32),
        ] + [pltpu.SemaphoreType.DMA] * (NI + NG + NO),
    )
    def k(x_hbm, nbr_hbm, out_hbm, idx_v, nbr_v, ctr_v, out_v, *sems):
        isem = sems[:NI]
        gsem = sems[NI:NI + NG]
        ssem = sems[NI + NG:]
        wid = lax.axis_index("s") * NC + lax.axis_index("c")
        n0 = wid * NPW

        def issue_idx(g, si):
            e0 = (n0 + g * G) * K
            pltpu.async_copy(nbr_hbm.at[pl.ds(e0, GK)], idx_v.at[si], isem[si])

        def wait_idx(si):
            pltpu.make_async_copy(
                nbr_hbm.at[pl.ds(0, GK)], idx_v.at[si], isem[si]).wait()

        def issue_gather(g, sg, si):
            nbase = n0 + g * G
            pltpu.async_copy(x_hbm.at[idx_v.at[si]], nbr_v.at[sg], gsem[sg])
            pltpu.async_copy(x_hbm.at[pl.ds(nbase, G)], ctr_v.at[sg], gsem[sg])

        def wait_gather(sg, si):
            pltpu.make_async_copy(
                x_hbm.at[idx_v.at[si]], nbr_v.at[sg], gsem[sg]).wait()
            pltpu.make_async_copy(
                x_hbm.at[pl.ds(0, G)], ctr_v.at[sg], gsem[sg]).wait()

        def issue_out(g, so):
            e0 = (n0 + g * G) * K
            pltpu.async_copy(out_v.at[so], out_hbm.at[pl.ds(e0, GK)], ssem[so])

        def wait_out(so):
            pltpu.make_async_copy(
                out_v.at[so], out_hbm.at[pl.ds(0, GK)], ssem[so]).wait()

        def compute(sg, so):
            def i_body(i, car):
                r0 = i * K
                cvecs = [ctr_v[sg, i, pl.ds(j * _L, _L)] for j in range(NJ)]
                for kk in range(K):
                    r = r0 + kk
                    for j in range(NJ):
                        nv = nbr_v[sg, r, pl.ds(j * _L, _L)]
                        out_v[so, r, pl.ds(j * _L, _L)] = cvecs[j]
                        out_v[so, r, pl.ds(D + j * _L, _L)] = nv - cvecs[j]
                return car

            lax.fori_loop(0, G, i_body, 0)

        issue_idx(0, 0)
        issue_idx(1, 1)
        wait_idx(0)
        issue_gather(0, 0, 0)

        def quad_body(gg, car):
            for u in range(NO):
                g2 = gg * NO + u

                @pl.when(g2 + 2 < n_groups)
                def _():
                    issue_idx(g2 + 2, (u + 2) % NI)

                @pl.when(g2 + 1 < n_groups)
                def _():
                    wait_idx((u + 1) % NI)
                    issue_gather(g2 + 1, (u + 1) % NG, (u + 1) % NI)

                wait_gather(u % NG, u % NI)

                @pl.when(g2 >= NO)
                def _():
                    wait_out(u)

                compute(u % NG, u)
                issue_out(g2, u)
            return car

        lax.fori_loop(0, n_groups // NO, quad_body, 0)
        for t in range(min(NO, n_groups)):
            wait_out((n_groups - 1 - t) % NO)

    return k(x2, nbr1)


def kernel(X_inputs, nn_idx):
    B, N, D = X_inputs.shape
    K = nn_idx.shape[-1]
    x2 = X_inputs.reshape(B * N, D)
    offs = (jnp.arange(B, dtype=jnp.int32) * N).reshape(B, 1, 1)
    nbr1 = (nn_idx.astype(jnp.int32) + offs).reshape(B * N * K)
    out = _edge_sc(x2, nbr1, B * N, D, K, 4)
    return out.reshape(B, N, K, 2 * D)


# speedup vs baseline: 1.0154x; 1.0154x over previous
import functools

import jax
import jax.numpy as jnp
from jax import lax
from jax.experimental import pallas as pl
from jax.experimental.pallas import tpu as pltpu
from jax.experimental.pallas import tpu_sc as plsc

_L = 16


@functools.partial(jax.jit, static_argnums=(2, 3, 4, 5))
def _edge_sc(x2, nbr1, BN, D, K, G):
    NC, NS = 2, 16
    NPW = BN // (NC * NS)
    NPC = BN // NC
    GK = G * K
    n_groups = NPW // G
    NI, NG, NO = 2, 2, 2
    NJ = D // _L

    mesh = plsc.VectorSubcoreMesh(core_axis_name="c", subcore_axis_name="s")

    @functools.partial(
        pl.kernel,
        mesh=mesh,
        out_type=jax.ShapeDtypeStruct((BN * K, 2 * D), jnp.float32),
        scratch_types=[
            pltpu.VMEM_SHARED((BN // NC, D), jnp.float32),
            pltpu.VMEM((NI, GK), jnp.int32),
            pltpu.VMEM((NG, GK, D), jnp.float32),
            pltpu.VMEM((NG, G, D), jnp.float32),
            pltpu.VMEM((NO, GK, 2 * D), jnp.float32),
        ] + [pltpu.SemaphoreType.DMA] * (1 + NI + NG + NO),
    )
    def k(x_hbm, nbr_hbm, out_hbm, xs, idx_v, nbr_v, ctr_v, out_v, *sems):
        stsem = sems[0]
        isem = sems[1:1 + NI]
        gsem = sems[1 + NI:1 + NI + NG]
        ssem = sems[1 + NI + NG:]
        cid = lax.axis_index("c")
        sid = lax.axis_index("s")
        wid = cid * NS + sid
        n0 = wid * NPW
        l0 = sid * NPW

        pltpu.async_copy(
            x_hbm.at[pl.ds(cid * NPC + sid * NPW, NPW)],
            xs.at[pl.ds(sid * NPW, NPW)], stsem).wait()
        plsc.subcore_barrier()

        def issue_idx(g, si):
            e0 = (n0 + g * G) * K
            pltpu.async_copy(nbr_hbm.at[pl.ds(e0, GK)], idx_v.at[si], isem[si])

        def wait_idx(si):
            pltpu.make_async_copy(
                nbr_hbm.at[pl.ds(0, GK)], idx_v.at[si], isem[si]).wait()

        def issue_gather(g, sg, si):
            lbase = l0 + g * G
            pltpu.async_copy(xs.at[idx_v.at[si]], nbr_v.at[sg], gsem[sg])
            pltpu.async_copy(xs.at[pl.ds(lbase, G)], ctr_v.at[sg], gsem[sg])

        def wait_gather(sg, si):
            pltpu.make_async_copy(
                xs.at[idx_v.at[si]], nbr_v.at[sg], gsem[sg]).wait()
            pltpu.make_async_copy(
                xs.at[pl.ds(0, G)], ctr_v.at[sg], gsem[sg]).wait()

        def issue_out(g, so):
            e0 = (n0 + g * G) * K
            pltpu.async_copy(out_v.at[so], out_hbm.at[pl.ds(e0, GK)], ssem[so])

        def wait_out(so):
            pltpu.make_async_copy(
                out_v.at[so], out_hbm.at[pl.ds(0, GK)], ssem[so]).wait()

        def compute(sg, so):
            def i_body(i, car):
                r0 = i * K
                cvecs = [ctr_v[sg, i, pl.ds(j * _L, _L)] for j in range(NJ)]

                def k_body(k4, car2):
                    for u in range(4):
                        r = r0 + k4 * 4 + u
                        for j in range(NJ):
                            nv = nbr_v[sg, r, pl.ds(j * _L, _L)]
                            out_v[so, r, pl.ds(j * _L, _L)] = cvecs[j]
                            out_v[so, r, pl.ds(D + j * _L, _L)] = nv - cvecs[j]
                    return car2

                return lax.fori_loop(0, K // 4, k_body, car)

            lax.fori_loop(0, G, i_body, 0)

        issue_idx(0, 0)
        issue_idx(1, 1)
        wait_idx(0)
        issue_gather(0, 0, 0)

        def duo_body(gg, car):
            for u in range(NO):
                g2 = gg * NO + u

                wait_gather(u, u)

                @pl.when(g2 + 2 < n_groups)
                def _():
                    issue_idx(g2 + 2, u)

                @pl.when(g2 + 1 < n_groups)
                def _():
                    wait_idx(1 - u)
                    issue_gather(g2 + 1, 1 - u, 1 - u)

                @pl.when(g2 >= NO)
                def _():
                    wait_out(u)

                compute(u, u)
                issue_out(g2, u)
            return car

        lax.fori_loop(0, n_groups // NO, duo_body, 0)
        for t in range(min(NO, n_groups)):
            wait_out((n_groups - 1 - t) % NO)

    return k(x2, nbr1)


def kernel(X_inputs, nn_idx):
    B, N, D = X_inputs.shape
    K = nn_idx.shape[-1]
    BN = B * N
    x2 = X_inputs.reshape(BN, D)
    offs = ((jnp.arange(B, dtype=jnp.int32) * N) % (BN // 2)).reshape(B, 1, 1)
    nbr1 = (nn_idx.astype(jnp.int32) + offs).reshape(BN * K)
    out = _edge_sc(x2, nbr1, BN, D, K, 4)
    return out.reshape(B, N, K, 2 * D)
